# baseline (speedup 1.0000x reference)
import functools

import jax
import jax.numpy as jnp
from jax import lax
from jax.experimental import pallas as pl
from jax.experimental.pallas import tpu as pltpu
from jax.experimental.pallas import tpu_sc as plsc

N = 10000
E = 50000
D = 512
Q = 4
DQ = D // Q
NT = 16
CK = 128
NCH = 25
E_PAD = NT * NCH * CK
N_ACC = 10240
STRIPE = N_ACC // NT
ZB = 32
BT = 1000


def _make_sc_l0():
    mesh = plsc.VectorSubcoreMesh(core_axis_name="c", subcore_axis_name="s")
    outs = tuple(jax.ShapeDtypeStruct((N_ACC, DQ), jnp.float32)
                 for _ in range(16))
    scratch = [
        pltpu.VMEM((NCH, CK), jnp.int32),
        pltpu.VMEM((NCH, CK), jnp.int32),
        pltpu.VMEM((NCH, CK), jnp.int32),
        pltpu.VMEM((NCH, CK), jnp.int32),
        pltpu.VMEM((NCH, CK), jnp.int32),
        pltpu.VMEM((NCH, CK), jnp.int32),
        pltpu.VMEM((CK, DQ), jnp.float32),
        pltpu.VMEM((ZB, DQ), jnp.float32),
        pltpu.VMEM_SHARED((N_ACC, DQ), jnp.float32),
        pltpu.SemaphoreType.DMA,
    ]

    @functools.partial(pl.kernel, out_type=outs, mesh=mesh,
                       scratch_types=scratch)
    def l0(t0, t1, t2, t3, e0, e1, e2, e3,
           src_e, dst_e, src_t, dst_t, src_v, dst_v, zrow_hbm, ones_hbm,
           se0, se1, se2, se3, st0, st1, st2, st3, sv0, sv1, sv2, sv3,
           c_e, c_ta, c_tb, c_v,
           srcv_e, srcv_t, srcv_v, dstv_e, dstv_t, dstv_v,
           rows_v, zb, acc, sem):
        c = lax.axis_index("c")
        t = lax.axis_index("s")
        sl = pl.ds(t * STRIPE, STRIPE)

        pltpu.sync_copy(src_e.at[t], srcv_e)
        pltpu.sync_copy(dst_e.at[t], dstv_e)
        pltpu.sync_copy(src_t.at[t], srcv_t)
        pltpu.sync_copy(dst_t.at[t], dstv_t)
        pltpu.sync_copy(src_v.at[t], srcv_v)
        pltpu.sync_copy(dst_v.at[t], dstv_v)
        pltpu.sync_copy(zrow_hbm, zb)

        def zero_acc():
            def zbody(k, carry):
                pltpu.sync_copy(zb, acc.at[pl.ds(t * STRIPE + k * ZB, ZB)])
                return carry

            lax.fori_loop(0, STRIPE // ZB, zbody, 0)

        def run_counts(dstv, out, lo, hi):
            zero_acc()
            pltpu.sync_copy(ones_hbm, rows_v)
            plsc.subcore_barrier()

            def body(j, carry):
                pltpu.sync_copy(rows_v, acc.at[dstv.at[j]], add=True)
                return carry

            lax.fori_loop(lo, hi, body, 0)
            plsc.subcore_barrier()
            pltpu.sync_copy(acc.at[sl], out.at[sl])
            plsc.subcore_barrier()

        def run_quarter(hq, srcv, dstv, sq):
            zero_acc()
            plsc.subcore_barrier()

            def body(j, carry):
                pltpu.async_copy(hq.at[srcv.at[j]], rows_v, sem).wait()
                pltpu.sync_copy(rows_v, acc.at[dstv.at[j]], add=True)
                return carry

            lax.fori_loop(0, NCH, body, 0)
            plsc.subcore_barrier()
            pltpu.sync_copy(acc.at[sl], sq.at[sl])
            plsc.subcore_barrier()

        @pl.when(c == 0)
        def _():
            run_counts(dstv_e, c_e, 0, NCH)
            run_counts(dstv_t, c_ta, 0, NCH // 2)
            run_quarter(e0, srcv_e, dstv_e, se0)
            run_quarter(e1, srcv_e, dstv_e, se1)
            run_quarter(t0, srcv_t, dstv_t, st0)
            run_quarter(t1, srcv_t, dstv_t, st1)
            run_quarter(t0, srcv_v, dstv_v, sv0)
            run_quarter(t1, srcv_v, dstv_v, sv1)

        @pl.when(c == 1)
        def _():
            run_counts(dstv_v, c_v, 0, NCH)
            run_counts(dstv_t, c_tb, NCH // 2, NCH)
            run_quarter(e2, srcv_e, dstv_e, se2)
            run_quarter(e3, srcv_e, dstv_e, se3)
            run_quarter(t2, srcv_t, dstv_t, st2)
            run_quarter(t3, srcv_t, dstv_t, st3)
            run_quarter(t2, srcv_v, dstv_v, sv2)
            run_quarter(t3, srcv_v, dstv_v, sv3)

    return l0


def _make_sc_l1():
    mesh = plsc.VectorSubcoreMesh(core_axis_name="c", subcore_axis_name="s")
    outs = tuple(jax.ShapeDtypeStruct((N_ACC, DQ), jnp.float32)
                 for _ in range(8))
    scratch = [
        pltpu.VMEM((NCH, CK), jnp.int32),
        pltpu.VMEM((NCH, CK), jnp.int32),
        pltpu.VMEM((NCH, CK), jnp.int32),
        pltpu.VMEM((NCH, CK), jnp.int32),
        pltpu.VMEM((CK, DQ), jnp.float32),
        pltpu.VMEM((ZB, DQ), jnp.float32),
        pltpu.VMEM_SHARED((N_ACC, DQ), jnp.float32),
        pltpu.SemaphoreType.DMA,
    ]

    @functools.partial(pl.kernel, out_type=outs, mesh=mesh,
                       scratch_types=scratch)
    def l1(t0, t1, t2, t3, e0, e1, e2, e3,
           src_e, dst_e, src_t, dst_t, zrow_hbm,
           se0, se1, se2, se3, st0, st1, st2, st3,
           srcv_e, srcv_t, dstv_e, dstv_t, rows_v, zb, acc, sem):
        c = lax.axis_index("c")
        t = lax.axis_index("s")
        sl = pl.ds(t * STRIPE, STRIPE)

        pltpu.sync_copy(src_e.at[t], srcv_e)
        pltpu.sync_copy(dst_e.at[t], dstv_e)
        pltpu.sync_copy(src_t.at[t], srcv_t)
        pltpu.sync_copy(dst_t.at[t], dstv_t)
        pltpu.sync_copy(zrow_hbm, zb)

        def run_quarter(hq, srcv, dstv, sq):
            def zbody(k, carry):
                pltpu.sync_copy(zb, acc.at[pl.ds(t * STRIPE + k * ZB, ZB)])
                return carry

            lax.fori_loop(0, STRIPE // ZB, zbody, 0)
            plsc.subcore_barrier()

            def body(j, carry):
                pltpu.async_copy(hq.at[srcv.at[j]], rows_v, sem).wait()
                pltpu.sync_copy(rows_v, acc.at[dstv.at[j]], add=True)
                return carry

            lax.fori_loop(0, NCH, body, 0)
            plsc.subcore_barrier()
            pltpu.sync_copy(acc.at[sl], sq.at[sl])
            plsc.subcore_barrier()

        @pl.when(c == 0)
        def _():
            run_quarter(e0, srcv_e, dstv_e, se0)
            run_quarter(e1, srcv_e, dstv_e, se1)
            run_quarter(t0, srcv_t, dstv_t, st0)
            run_quarter(t1, srcv_t, dstv_t, st1)

        @pl.when(c == 1)
        def _():
            run_quarter(e2, srcv_e, dstv_e, se2)
            run_quarter(e3, srcv_e, dstv_e, se3)
            run_quarter(t2, srcv_t, dstv_t, st2)
            run_quarter(t3, srcv_t, dstv_t, st3)

    return l1


@functools.lru_cache(maxsize=None)
def _sc_l0():
    return _make_sc_l0()


@functools.lru_cache(maxsize=None)
def _sc_l1():
    return _make_sc_l1()


def _mean_mm(qrefs, cnt_refs, w_ref, b_ref):
    s = jnp.concatenate([q[...] for q in qrefs], axis=1)
    c = sum(cr[:, 0:1] for cr in cnt_refs)
    r = 1.0 / jnp.maximum(c, 1.0)
    m = (c > 0).astype(jnp.float32)
    h = lax.dot_general(s * r, w_ref[...], (((1,), (0,)), ((), ())),
                        preferred_element_type=jnp.float32,
                        precision=lax.Precision.DEFAULT)
    return h + m * b_ref[...]


def _leaky(x):
    return jnp.where(x >= 0, x, 0.01 * x)


def _l0_body(se0, se1, se2, se3, st0, st1, st2, st3, sv0, sv1, sv2, sv3,
             ce, cta, ctb, cv, w_e, w_t, w_v, b_e, b_t, b_v,
             ot0, ot1, ot2, ot3, oe0, oe1, oe2, oe3):
    ht = _leaky(_mean_mm((se0, se1, se2, se3), (ce,), w_e, b_e)
                + _mean_mm((st0, st1, st2, st3), (cta, ctb), w_t, b_t))
    he = _leaky(_mean_mm((sv0, sv1, sv2, sv3), (cv,), w_v, b_v))
    for q, o in enumerate((ot0, ot1, ot2, ot3)):
        o[...] = ht[:, q * DQ:(q + 1) * DQ]
    for q, o in enumerate((oe0, oe1, oe2, oe3)):
        o[...] = he[:, q * DQ:(q + 1) * DQ]


def _l1_body(se0, se1, se2, se3, st0, st1, st2, st3,
             ce, cta, ctb, w_e, w_t, b_e, b_t, w_out, b_out, out):
    h = (_mean_mm((se0, se1, se2, se3), (ce,), w_e, b_e)
         + _mean_mm((st0, st1, st2, st3), (cta, ctb), w_t, b_t))
    out[...] = lax.dot_general(h, w_out[...], (((1,), (0,)), ((), ())),
                               preferred_element_type=jnp.float32,
                               precision=lax.Precision.DEFAULT) + b_out[...]


def _feat_spec():
    return pl.BlockSpec((BT, DQ), lambda i: (i, 0))


def _cnt_spec():
    return pl.BlockSpec((BT, DQ), lambda i: (i, 0))


def _full_spec(shape):
    return pl.BlockSpec(shape, lambda i: tuple(0 for _ in shape))


def _layer0_tc(se, st, sv, ce, cta, ctb, cv, w_e, b_e, w_t, b_t, w_v, b_v):
    grid = (N // BT,)
    in_specs = ([_feat_spec() for _ in range(12)]
                + [_cnt_spec() for _ in range(4)]
                + [_full_spec((D, D)) for _ in range(3)]
                + [_full_spec((1, D)) for _ in range(3)])
    out_specs = [_feat_spec() for _ in range(8)]
    outs = [jax.ShapeDtypeStruct((N, DQ), jnp.float32) for _ in range(8)]
    res = pl.pallas_call(
        _l0_body, grid=grid, in_specs=in_specs, out_specs=out_specs,
        out_shape=outs)(
            *se, *st, *sv, ce, cta, ctb, cv, w_e, w_t, w_v,
            b_e.reshape(1, D), b_t.reshape(1, D), b_v.reshape(1, D))
    return res[:4], res[4:]


def _layer1_tc(se, st, ce, cta, ctb, w_e, b_e, w_t, b_t, w_out, b_out):
    grid = (N // BT,)
    in_specs = ([_feat_spec() for _ in range(8)]
                + [_cnt_spec() for _ in range(3)]
                + [_full_spec((D, D)) for _ in range(2)]
                + [_full_spec((1, D)) for _ in range(2)]
                + [_full_spec((D, 128)), _full_spec((1, 128))])
    out_spec = pl.BlockSpec((BT, 128), lambda i: (i, 0))
    return pl.pallas_call(
        _l1_body, grid=grid, in_specs=in_specs, out_specs=out_spec,
        out_shape=jax.ShapeDtypeStruct((N, 128), jnp.float32))(
            *se, *st, ce, cta, ctb, w_e, w_t,
            b_e.reshape(1, D), b_t.reshape(1, D),
            w_out, b_out.reshape(1, 128))


def _quarters(x):
    return tuple(x[:, q * DQ:(q + 1) * DQ] for q in range(Q))


def _prep_edges(e):
    src = jnp.concatenate(
        [e[0].astype(jnp.int32), jnp.zeros((E_PAD - E,), jnp.int32)])
    dst = jnp.concatenate(
        [e[1].astype(jnp.int32), jnp.full((E_PAD - E,), N, jnp.int32)])
    return src.reshape(NT, NCH, CK), dst.reshape(NT, NCH, CK)


def kernel(features, embed_entity, edge_e2t, edge_t2e, edge_t2t,
           W0_e2t, b0_e2t, W0_t2e, b0_t2e, W0_t2t, b0_t2t,
           W1_e2t, b1_e2t, W1_t2e, b1_t2e, W1_t2t, b1_t2t,
           Wout, bout):
    ft = _quarters(features)
    fe = _quarters(embed_entity)
    src_e2t, dst_e2t = _prep_edges(edge_e2t)
    src_t2e, dst_t2e = _prep_edges(edge_t2e)
    src_t2t, dst_t2t = _prep_edges(edge_t2t)
    zrow = jnp.zeros((ZB, DQ), jnp.float32)
    ones = jnp.ones((CK, DQ), jnp.float32)

    l0_out = _sc_l0()(*ft, *fe,
                      src_e2t, dst_e2t, src_t2t, dst_t2t, src_t2e, dst_t2e,
                      zrow, ones)
    s_e2t = l0_out[0:4]
    s_t2t = l0_out[4:8]
    s_t2e = l0_out[8:12]
    cnt_e2t, cnt_t2t_a, cnt_t2t_b, cnt_t2e = l0_out[12:16]

    ht_q, he_q = _layer0_tc(s_e2t, s_t2t, s_t2e,
                            cnt_e2t, cnt_t2t_a, cnt_t2t_b, cnt_t2e,
                            W0_e2t, b0_e2t, W0_t2t, b0_t2t, W0_t2e, b0_t2e)

    l1_out = _sc_l1()(*ht_q, *he_q, src_e2t, dst_e2t, src_t2t, dst_t2t,
                      zrow)
    s1_e2t = l1_out[0:4]
    s1_t2t = l1_out[4:8]

    return _layer1_tc(s1_e2t, s1_t2t, cnt_e2t, cnt_t2t_a, cnt_t2t_b,
                      W1_e2t, b1_e2t, W1_t2t, b1_t2t, Wout, bout)


# speedup vs baseline: 1.0028x; 1.0028x over previous
import functools

import jax
import jax.numpy as jnp
from jax import lax
from jax.experimental import pallas as pl
from jax.experimental.pallas import tpu as pltpu
from jax.experimental.pallas import tpu_sc as plsc

N = 10000
E = 50000
D = 512
Q = 4
DQ = D // Q
NT = 16
CK = 128
NCH = 25
E_PAD = NT * NCH * CK
N_ACC = 10240
STRIPE = N_ACC // NT
ZB = 32
BT = 1000


def _make_sc_l0():
    mesh = plsc.VectorSubcoreMesh(core_axis_name="c", subcore_axis_name="s")
    outs = tuple(jax.ShapeDtypeStruct((N_ACC, DQ), jnp.float32)
                 for _ in range(15))
    scratch = [
        pltpu.VMEM((NCH, CK), jnp.int32),
        pltpu.VMEM((NCH, CK), jnp.int32),
        pltpu.VMEM((NCH, CK), jnp.int32),
        pltpu.VMEM((NCH, CK), jnp.int32),
        pltpu.VMEM((NCH, CK), jnp.int32),
        pltpu.VMEM((NCH, CK), jnp.int32),
        pltpu.VMEM((CK, DQ), jnp.float32),
        pltpu.VMEM((ZB, DQ), jnp.float32),
        pltpu.VMEM_SHARED((N_ACC, DQ), jnp.float32),
        pltpu.SemaphoreType.DMA,
    ]

    @functools.partial(pl.kernel, out_type=outs, mesh=mesh,
                       scratch_types=scratch)
    def l0(t0, t1, t2, t3, e0, e1, e2, e3,
           src_e, dst_e, src_t, dst_t, src_v, dst_v, zrow_hbm, ones_hbm,
           se0, se1, se2, se3, st0, st1, st2, st3, sv0, sv1, sv2, sv3,
           c_e, c_t, c_v,
           srcv_e, srcv_t, srcv_v, dstv_e, dstv_t, dstv_v,
           rows_v, zb, acc, sem):
        c = lax.axis_index("c")
        t = lax.axis_index("s")
        sl = pl.ds(t * STRIPE, STRIPE)

        pltpu.sync_copy(src_e.at[t], srcv_e)
        pltpu.sync_copy(dst_e.at[t], dstv_e)
        pltpu.sync_copy(src_t.at[t], srcv_t)
        pltpu.sync_copy(dst_t.at[t], dstv_t)
        pltpu.sync_copy(src_v.at[t], srcv_v)
        pltpu.sync_copy(dst_v.at[t], dstv_v)
        pltpu.sync_copy(zrow_hbm, zb)

        def zero_acc():
            def zbody(k, carry):
                pltpu.sync_copy(zb, acc.at[pl.ds(t * STRIPE + k * ZB, ZB)])
                return carry

            lax.fori_loop(0, STRIPE // ZB, zbody, 0)

        def run_counts(dstv, out, lo, hi):
            zero_acc()
            pltpu.sync_copy(ones_hbm, rows_v)
            plsc.subcore_barrier()

            def body(j, carry):
                pltpu.sync_copy(rows_v, acc.at[dstv.at[j]], add=True)
                return carry

            lax.fori_loop(lo, hi, body, 0)
            plsc.subcore_barrier()
            pltpu.sync_copy(acc.at[sl], out.at[sl])
            plsc.subcore_barrier()

        def run_quarter(hq, srcv, dstv, sq):
            zero_acc()
            plsc.subcore_barrier()

            def body(j, carry):
                pltpu.async_copy(hq.at[srcv.at[j]], rows_v, sem).wait()
                pltpu.sync_copy(rows_v, acc.at[dstv.at[j]], add=True)
                return carry

            lax.fori_loop(0, NCH, body, 0)
            plsc.subcore_barrier()
            pltpu.sync_copy(acc.at[sl], sq.at[sl])
            plsc.subcore_barrier()

        @pl.when(c == 0)
        def _():
            run_counts(dstv_e, c_e, 0, NCH)
            run_counts(dstv_t, c_t, 0, NCH)
            run_quarter(e0, srcv_e, dstv_e, se0)
            run_quarter(e1, srcv_e, dstv_e, se1)
            run_quarter(t0, srcv_t, dstv_t, st0)
            run_quarter(t1, srcv_t, dstv_t, st1)
            run_quarter(t0, srcv_v, dstv_v, sv0)
            run_quarter(t1, srcv_v, dstv_v, sv1)

        @pl.when(c == 1)
        def _():
            run_counts(dstv_v, c_v, 0, NCH)
            run_quarter(e2, srcv_e, dstv_e, se2)
            run_quarter(e3, srcv_e, dstv_e, se3)
            run_quarter(t2, srcv_t, dstv_t, st2)
            run_quarter(t3, srcv_t, dstv_t, st3)
            run_quarter(t2, srcv_v, dstv_v, sv2)
            run_quarter(t3, srcv_v, dstv_v, sv3)

    return l0


def _make_sc_l1():
    mesh = plsc.VectorSubcoreMesh(core_axis_name="c", subcore_axis_name="s")
    outs = tuple(jax.ShapeDtypeStruct((N_ACC, DQ), jnp.float32)
                 for _ in range(8))
    scratch = [
        pltpu.VMEM((NCH, CK), jnp.int32),
        pltpu.VMEM((NCH, CK), jnp.int32),
        pltpu.VMEM((NCH, CK), jnp.int32),
        pltpu.VMEM((NCH, CK), jnp.int32),
        pltpu.VMEM((CK, DQ), jnp.float32),
        pltpu.VMEM((ZB, DQ), jnp.float32),
        pltpu.VMEM_SHARED((N_ACC, DQ), jnp.float32),
        pltpu.SemaphoreType.DMA,
    ]

    @functools.partial(pl.kernel, out_type=outs, mesh=mesh,
                       scratch_types=scratch)
    def l1(t0, t1, t2, t3, e0, e1, e2, e3,
           src_e, dst_e, src_t, dst_t, zrow_hbm,
           se0, se1, se2, se3, st0, st1, st2, st3,
           srcv_e, srcv_t, dstv_e, dstv_t, rows_v, zb, acc, sem):
        c = lax.axis_index("c")
        t = lax.axis_index("s")
        sl = pl.ds(t * STRIPE, STRIPE)

        pltpu.sync_copy(src_e.at[t], srcv_e)
        pltpu.sync_copy(dst_e.at[t], dstv_e)
        pltpu.sync_copy(src_t.at[t], srcv_t)
        pltpu.sync_copy(dst_t.at[t], dstv_t)
        pltpu.sync_copy(zrow_hbm, zb)

        def run_quarter(hq, srcv, dstv, sq):
            def zbody(k, carry):
                pltpu.sync_copy(zb, acc.at[pl.ds(t * STRIPE + k * ZB, ZB)])
                return carry

            lax.fori_loop(0, STRIPE // ZB, zbody, 0)
            plsc.subcore_barrier()

            def body(j, carry):
                pltpu.async_copy(hq.at[srcv.at[j]], rows_v, sem).wait()
                pltpu.sync_copy(rows_v, acc.at[dstv.at[j]], add=True)
                return carry

            lax.fori_loop(0, NCH, body, 0)
            plsc.subcore_barrier()
            pltpu.sync_copy(acc.at[sl], sq.at[sl])
            plsc.subcore_barrier()

        @pl.when(c == 0)
        def _():
            run_quarter(e0, srcv_e, dstv_e, se0)
            run_quarter(e1, srcv_e, dstv_e, se1)
            run_quarter(t0, srcv_t, dstv_t, st0)
            run_quarter(t1, srcv_t, dstv_t, st1)

        @pl.when(c == 1)
        def _():
            run_quarter(e2, srcv_e, dstv_e, se2)
            run_quarter(e3, srcv_e, dstv_e, se3)
            run_quarter(t2, srcv_t, dstv_t, st2)
            run_quarter(t3, srcv_t, dstv_t, st3)

    return l1


@functools.lru_cache(maxsize=None)
def _sc_l0():
    return _make_sc_l0()


@functools.lru_cache(maxsize=None)
def _sc_l1():
    return _make_sc_l1()


def _mean_mm(qrefs, cnt_refs, w_ref, b_ref):
    s = jnp.concatenate([q[...] for q in qrefs], axis=1)
    c = sum(cr[:, 0:1] for cr in cnt_refs)
    r = 1.0 / jnp.maximum(c, 1.0)
    m = (c > 0).astype(jnp.float32)
    h = lax.dot_general(s * r, w_ref[...], (((1,), (0,)), ((), ())),
                        preferred_element_type=jnp.float32,
                        precision=lax.Precision.DEFAULT)
    return h + m * b_ref[...]


def _leaky(x):
    return jnp.where(x >= 0, x, 0.01 * x)


def _l0_body(se0, se1, se2, se3, st0, st1, st2, st3, sv0, sv1, sv2, sv3,
             ce, ct, cv, w_e, w_t, w_v, b_e, b_t, b_v,
             ot0, ot1, ot2, ot3, oe0, oe1, oe2, oe3):
    ht = _leaky(_mean_mm((se0, se1, se2, se3), (ce,), w_e, b_e)
                + _mean_mm((st0, st1, st2, st3), (ct,), w_t, b_t))
    he = _leaky(_mean_mm((sv0, sv1, sv2, sv3), (cv,), w_v, b_v))
    for q, o in enumerate((ot0, ot1, ot2, ot3)):
        o[...] = ht[:, q * DQ:(q + 1) * DQ]
    for q, o in enumerate((oe0, oe1, oe2, oe3)):
        o[...] = he[:, q * DQ:(q + 1) * DQ]


def _l1_body(se0, se1, se2, se3, st0, st1, st2, st3,
             ce, ct, w_e, w_t, b_e, b_t, w_out, b_out, out):
    h = (_mean_mm((se0, se1, se2, se3), (ce,), w_e, b_e)
         + _mean_mm((st0, st1, st2, st3), (ct,), w_t, b_t))
    out[...] = lax.dot_general(h, w_out[...], (((1,), (0,)), ((), ())),
                               preferred_element_type=jnp.float32,
                               precision=lax.Precision.DEFAULT) + b_out[...]


def _feat_spec():
    return pl.BlockSpec((BT, DQ), lambda i: (i, 0))


def _cnt_spec():
    return pl.BlockSpec((BT, DQ), lambda i: (i, 0))


def _full_spec(shape):
    return pl.BlockSpec(shape, lambda i: tuple(0 for _ in shape))


def _layer0_tc(se, st, sv, ce, ct, cv, w_e, b_e, w_t, b_t, w_v, b_v):
    grid = (N // BT,)
    in_specs = ([_feat_spec() for _ in range(12)]
                + [_cnt_spec() for _ in range(3)]
                + [_full_spec((D, D)) for _ in range(3)]
                + [_full_spec((1, D)) for _ in range(3)])
    out_specs = [_feat_spec() for _ in range(8)]
    outs = [jax.ShapeDtypeStruct((N, DQ), jnp.float32) for _ in range(8)]
    res = pl.pallas_call(
        _l0_body, grid=grid, in_specs=in_specs, out_specs=out_specs,
        out_shape=outs)(
            *se, *st, *sv, ce, ct, cv, w_e, w_t, w_v,
            b_e.reshape(1, D), b_t.reshape(1, D), b_v.reshape(1, D))
    return res[:4], res[4:]


def _layer1_tc(se, st, ce, ct, w_e, b_e, w_t, b_t, w_out, b_out):
    grid = (N // BT,)
    in_specs = ([_feat_spec() for _ in range(8)]
                + [_cnt_spec() for _ in range(2)]
                + [_full_spec((D, D)) for _ in range(2)]
                + [_full_spec((1, D)) for _ in range(2)]
                + [_full_spec((D, 128)), _full_spec((1, 128))])
    out_spec = pl.BlockSpec((BT, 128), lambda i: (i, 0))
    return pl.pallas_call(
        _l1_body, grid=grid, in_specs=in_specs, out_specs=out_spec,
        out_shape=jax.ShapeDtypeStruct((N, 128), jnp.float32))(
            *se, *st, ce, ct, w_e, w_t,
            b_e.reshape(1, D), b_t.reshape(1, D),
            w_out, b_out.reshape(1, 128))


def _quarters(x):
    return tuple(x[:, q * DQ:(q + 1) * DQ] for q in range(Q))


def _prep_edges(e):
    src = jnp.concatenate(
        [e[0].astype(jnp.int32), jnp.zeros((E_PAD - E,), jnp.int32)])
    dst = jnp.concatenate(
        [e[1].astype(jnp.int32), jnp.full((E_PAD - E,), N, jnp.int32)])
    return src.reshape(NT, NCH, CK), dst.reshape(NT, NCH, CK)


def kernel(features, embed_entity, edge_e2t, edge_t2e, edge_t2t,
           W0_e2t, b0_e2t, W0_t2e, b0_t2e, W0_t2t, b0_t2t,
           W1_e2t, b1_e2t, W1_t2e, b1_t2e, W1_t2t, b1_t2t,
           Wout, bout):
    ft = _quarters(features)
    fe = _quarters(embed_entity)
    src_e2t, dst_e2t = _prep_edges(edge_e2t)
    src_t2e, dst_t2e = _prep_edges(edge_t2e)
    src_t2t, dst_t2t = _prep_edges(edge_t2t)
    zrow = jnp.zeros((ZB, DQ), jnp.float32)
    ones = jnp.ones((CK, DQ), jnp.float32)

    l0_out = _sc_l0()(*ft, *fe,
                      src_e2t, dst_e2t, src_t2t, dst_t2t, src_t2e, dst_t2e,
                      zrow, ones)
    s_e2t = l0_out[0:4]
    s_t2t = l0_out[4:8]
    s_t2e = l0_out[8:12]
    cnt_e2t, cnt_t2t, cnt_t2e = l0_out[12:15]

    ht_q, he_q = _layer0_tc(s_e2t, s_t2t, s_t2e,
                            cnt_e2t, cnt_t2t, cnt_t2e,
                            W0_e2t, b0_e2t, W0_t2t, b0_t2t, W0_t2e, b0_t2e)

    l1_out = _sc_l1()(*ht_q, *he_q, src_e2t, dst_e2t, src_t2t, dst_t2t,
                      zrow)
    s1_e2t = l1_out[0:4]
    s1_t2t = l1_out[4:8]

    return _layer1_tc(s1_e2t, s1_t2t, cnt_e2t, cnt_t2t,
                      W1_e2t, b1_e2t, W1_t2t, b1_t2t, Wout, bout)


# speedup vs baseline: 1.0062x; 1.0033x over previous
import functools

import jax
import jax.numpy as jnp
from jax import lax
from jax.experimental import pallas as pl
from jax.experimental.pallas import tpu as pltpu
from jax.experimental.pallas import tpu_sc as plsc

N = 10000
E = 50000
D = 512
Q = 4
DQ = D // Q
NT = 16
CK = 128
NCH = 25
E_PAD = NT * NCH * CK
N_ACC = 10240
STRIPE = N_ACC // NT
ZB = 32
BT = 1000


def _make_sc_l0():
    mesh = plsc.VectorSubcoreMesh(core_axis_name="c", subcore_axis_name="s")
    outs = tuple(jax.ShapeDtypeStruct((N_ACC, DQ), jnp.float32)
                 for _ in range(15))
    scratch = [
        pltpu.VMEM((NCH, CK), jnp.int32),
        pltpu.VMEM((NCH, CK), jnp.int32),
        pltpu.VMEM((NCH, CK), jnp.int32),
        pltpu.VMEM((NCH, CK), jnp.int32),
        pltpu.VMEM((NCH, CK), jnp.int32),
        pltpu.VMEM((NCH, CK), jnp.int32),
        pltpu.VMEM((CK, DQ), jnp.float32),
        pltpu.VMEM((ZB, DQ), jnp.float32),
        pltpu.VMEM_SHARED((N_ACC, DQ), jnp.float32),
        pltpu.SemaphoreType.DMA,
    ]

    @functools.partial(pl.kernel, out_type=outs, mesh=mesh,
                       scratch_types=scratch)
    def l0(t0, t1, t2, t3, e0, e1, e2, e3,
           src_e, dst_e, src_t, dst_t, src_v, dst_v, zrow_hbm, ones_hbm,
           se0, se1, se2, se3, st0, st1, st2, st3, sv0, sv1, sv2, sv3,
           c_e, c_t, c_v,
           srcv_e, srcv_t, srcv_v, dstv_e, dstv_t, dstv_v,
           rows_v, zb, acc, sem):
        c = lax.axis_index("c")
        t = lax.axis_index("s")
        sl = pl.ds(t * STRIPE, STRIPE)

        pltpu.sync_copy(src_e.at[t], srcv_e)
        pltpu.sync_copy(dst_e.at[t], dstv_e)
        pltpu.sync_copy(src_t.at[t], srcv_t)
        pltpu.sync_copy(dst_t.at[t], dstv_t)
        pltpu.sync_copy(src_v.at[t], srcv_v)
        pltpu.sync_copy(dst_v.at[t], dstv_v)
        pltpu.sync_copy(zrow_hbm, zb)
        pltpu.sync_copy(ones_hbm, rows_v)

        def zero_acc():
            def zbody(k, carry):
                pltpu.sync_copy(zb, acc.at[pl.ds(t * STRIPE + k * ZB, ZB)])
                return carry

            lax.fori_loop(0, STRIPE // ZB, zbody, 0)

        def run_counts(dstv, out, lo, hi):
            zero_acc()
            plsc.subcore_barrier()

            def body(j, carry):
                pltpu.sync_copy(rows_v, acc.at[dstv.at[j]], add=True)
                return carry

            lax.fori_loop(lo, hi, body, 0)
            plsc.subcore_barrier()
            pltpu.sync_copy(acc.at[sl], out.at[sl])

        def run_quarter(hq, srcv, dstv, sq):
            zero_acc()
            plsc.subcore_barrier()

            def body(j, carry):
                pltpu.async_copy(hq.at[srcv.at[j]], rows_v, sem).wait()
                pltpu.sync_copy(rows_v, acc.at[dstv.at[j]], add=True)
                return carry

            lax.fori_loop(0, NCH, body, 0)
            plsc.subcore_barrier()
            pltpu.sync_copy(acc.at[sl], sq.at[sl])

        @pl.when(c == 0)
        def _():
            run_counts(dstv_e, c_e, 0, NCH)
            run_counts(dstv_t, c_t, 0, NCH)
            run_quarter(e0, srcv_e, dstv_e, se0)
            run_quarter(e1, srcv_e, dstv_e, se1)
            run_quarter(t0, srcv_t, dstv_t, st0)
            run_quarter(t1, srcv_t, dstv_t, st1)
            run_quarter(t0, srcv_v, dstv_v, sv0)
            run_quarter(t1, srcv_v, dstv_v, sv1)

        @pl.when(c == 1)
        def _():
            run_counts(dstv_v, c_v, 0, NCH)
            run_quarter(e2, srcv_e, dstv_e, se2)
            run_quarter(e3, srcv_e, dstv_e, se3)
            run_quarter(t2, srcv_t, dstv_t, st2)
            run_quarter(t3, srcv_t, dstv_t, st3)
            run_quarter(t2, srcv_v, dstv_v, sv2)
            run_quarter(t3, srcv_v, dstv_v, sv3)

    return l0


def _make_sc_l1():
    mesh = plsc.VectorSubcoreMesh(core_axis_name="c", subcore_axis_name="s")
    outs = tuple(jax.ShapeDtypeStruct((N_ACC, DQ), jnp.float32)
                 for _ in range(8))
    scratch = [
        pltpu.VMEM((NCH, CK), jnp.int32),
        pltpu.VMEM((NCH, CK), jnp.int32),
        pltpu.VMEM((NCH, CK), jnp.int32),
        pltpu.VMEM((NCH, CK), jnp.int32),
        pltpu.VMEM((CK, DQ), jnp.float32),
        pltpu.VMEM((ZB, DQ), jnp.float32),
        pltpu.VMEM_SHARED((N_ACC, DQ), jnp.float32),
        pltpu.SemaphoreType.DMA,
    ]

    @functools.partial(pl.kernel, out_type=outs, mesh=mesh,
                       scratch_types=scratch)
    def l1(t0, t1, t2, t3, e0, e1, e2, e3,
           src_e, dst_e, src_t, dst_t, zrow_hbm,
           se0, se1, se2, se3, st0, st1, st2, st3,
           srcv_e, srcv_t, dstv_e, dstv_t, rows_v, zb, acc, sem):
        c = lax.axis_index("c")
        t = lax.axis_index("s")
        sl = pl.ds(t * STRIPE, STRIPE)

        pltpu.sync_copy(src_e.at[t], srcv_e)
        pltpu.sync_copy(dst_e.at[t], dstv_e)
        pltpu.sync_copy(src_t.at[t], srcv_t)
        pltpu.sync_copy(dst_t.at[t], dstv_t)
        pltpu.sync_copy(zrow_hbm, zb)

        def run_quarter(hq, srcv, dstv, sq):
            def zbody(k, carry):
                pltpu.sync_copy(zb, acc.at[pl.ds(t * STRIPE + k * ZB, ZB)])
                return carry

            lax.fori_loop(0, STRIPE // ZB, zbody, 0)
            plsc.subcore_barrier()

            def body(j, carry):
                pltpu.async_copy(hq.at[srcv.at[j]], rows_v, sem).wait()
                pltpu.sync_copy(rows_v, acc.at[dstv.at[j]], add=True)
                return carry

            lax.fori_loop(0, NCH, body, 0)
            plsc.subcore_barrier()
            pltpu.sync_copy(acc.at[sl], sq.at[sl])

        @pl.when(c == 0)
        def _():
            run_quarter(e0, srcv_e, dstv_e, se0)
            run_quarter(e1, srcv_e, dstv_e, se1)
            run_quarter(t0, srcv_t, dstv_t, st0)
            run_quarter(t1, srcv_t, dstv_t, st1)

        @pl.when(c == 1)
        def _():
            run_quarter(e2, srcv_e, dstv_e, se2)
            run_quarter(e3, srcv_e, dstv_e, se3)
            run_quarter(t2, srcv_t, dstv_t, st2)
            run_quarter(t3, srcv_t, dstv_t, st3)

    return l1


@functools.lru_cache(maxsize=None)
def _sc_l0():
    return _make_sc_l0()


@functools.lru_cache(maxsize=None)
def _sc_l1():
    return _make_sc_l1()


def _mean_mm(qrefs, cnt_refs, w_ref, b_ref):
    s = jnp.concatenate([q[...] for q in qrefs], axis=1)
    c = sum(cr[:, 0:1] for cr in cnt_refs)
    r = 1.0 / jnp.maximum(c, 1.0)
    m = (c > 0).astype(jnp.float32)
    h = lax.dot_general(s * r, w_ref[...], (((1,), (0,)), ((), ())),
                        preferred_element_type=jnp.float32,
                        precision=lax.Precision.DEFAULT)
    return h + m * b_ref[...]


def _leaky(x):
    return jnp.where(x >= 0, x, 0.01 * x)


def _l0_body(se0, se1, se2, se3, st0, st1, st2, st3, sv0, sv1, sv2, sv3,
             ce, ct, cv, w_e, w_t, w_v, b_e, b_t, b_v,
             ot0, ot1, ot2, ot3, oe0, oe1, oe2, oe3):
    ht = _leaky(_mean_mm((se0, se1, se2, se3), (ce,), w_e, b_e)
                + _mean_mm((st0, st1, st2, st3), (ct,), w_t, b_t))
    he = _leaky(_mean_mm((sv0, sv1, sv2, sv3), (cv,), w_v, b_v))
    for q, o in enumerate((ot0, ot1, ot2, ot3)):
        o[...] = ht[:, q * DQ:(q + 1) * DQ]
    for q, o in enumerate((oe0, oe1, oe2, oe3)):
        o[...] = he[:, q * DQ:(q + 1) * DQ]


def _l1_body(se0, se1, se2, se3, st0, st1, st2, st3,
             ce, ct, w_e, w_t, b_e, b_t, w_out, b_out, out):
    h = (_mean_mm((se0, se1, se2, se3), (ce,), w_e, b_e)
         + _mean_mm((st0, st1, st2, st3), (ct,), w_t, b_t))
    out[...] = lax.dot_general(h, w_out[...], (((1,), (0,)), ((), ())),
                               preferred_element_type=jnp.float32,
                               precision=lax.Precision.DEFAULT) + b_out[...]


def _feat_spec():
    return pl.BlockSpec((BT, DQ), lambda i: (i, 0))


def _cnt_spec():
    return pl.BlockSpec((BT, DQ), lambda i: (i, 0))


def _full_spec(shape):
    return pl.BlockSpec(shape, lambda i: tuple(0 for _ in shape))


def _layer0_tc(se, st, sv, ce, ct, cv, w_e, b_e, w_t, b_t, w_v, b_v):
    grid = (N // BT,)
    in_specs = ([_feat_spec() for _ in range(12)]
                + [_cnt_spec() for _ in range(3)]
                + [_full_spec((D, D)) for _ in range(3)]
                + [_full_spec((1, D)) for _ in range(3)])
    out_specs = [_feat_spec() for _ in range(8)]
    outs = [jax.ShapeDtypeStruct((N, DQ), jnp.float32) for _ in range(8)]
    res = pl.pallas_call(
        _l0_body, grid=grid, in_specs=in_specs, out_specs=out_specs,
        out_shape=outs)(
            *se, *st, *sv, ce, ct, cv, w_e, w_t, w_v,
            b_e.reshape(1, D), b_t.reshape(1, D), b_v.reshape(1, D))
    return res[:4], res[4:]


def _layer1_tc(se, st, ce, ct, w_e, b_e, w_t, b_t, w_out, b_out):
    grid = (N // BT,)
    in_specs = ([_feat_spec() for _ in range(8)]
                + [_cnt_spec() for _ in range(2)]
                + [_full_spec((D, D)) for _ in range(2)]
                + [_full_spec((1, D)) for _ in range(2)]
                + [_full_spec((D, 128)), _full_spec((1, 128))])
    out_spec = pl.BlockSpec((BT, 128), lambda i: (i, 0))
    return pl.pallas_call(
        _l1_body, grid=grid, in_specs=in_specs, out_specs=out_spec,
        out_shape=jax.ShapeDtypeStruct((N, 128), jnp.float32))(
            *se, *st, ce, ct, w_e, w_t,
            b_e.reshape(1, D), b_t.reshape(1, D),
            w_out, b_out.reshape(1, 128))


def _quarters(x):
    return tuple(x[:, q * DQ:(q + 1) * DQ] for q in range(Q))


def _prep_edges(e):
    src = jnp.concatenate(
        [e[0].astype(jnp.int32), jnp.zeros((E_PAD - E,), jnp.int32)])
    dst = jnp.concatenate(
        [e[1].astype(jnp.int32), jnp.full((E_PAD - E,), N, jnp.int32)])
    return src.reshape(NT, NCH, CK), dst.reshape(NT, NCH, CK)


def kernel(features, embed_entity, edge_e2t, edge_t2e, edge_t2t,
           W0_e2t, b0_e2t, W0_t2e, b0_t2e, W0_t2t, b0_t2t,
           W1_e2t, b1_e2t, W1_t2e, b1_t2e, W1_t2t, b1_t2t,
           Wout, bout):
    ft = _quarters(features)
    fe = _quarters(embed_entity)
    src_e2t, dst_e2t = _prep_edges(edge_e2t)
    src_t2e, dst_t2e = _prep_edges(edge_t2e)
    src_t2t, dst_t2t = _prep_edges(edge_t2t)
    zrow = jnp.zeros((ZB, DQ), jnp.float32)
    ones = jnp.ones((CK, DQ), jnp.float32)

    l0_out = _sc_l0()(*ft, *fe,
                      src_e2t, dst_e2t, src_t2t, dst_t2t, src_t2e, dst_t2e,
                      zrow, ones)
    s_e2t = l0_out[0:4]
    s_t2t = l0_out[4:8]
    s_t2e = l0_out[8:12]
    cnt_e2t, cnt_t2t, cnt_t2e = l0_out[12:15]

    ht_q, he_q = _layer0_tc(s_e2t, s_t2t, s_t2e,
                            cnt_e2t, cnt_t2t, cnt_t2e,
                            W0_e2t, b0_e2t, W0_t2t, b0_t2t, W0_t2e, b0_t2e)

    l1_out = _sc_l1()(*ht_q, *he_q, src_e2t, dst_e2t, src_t2t, dst_t2t,
                      zrow)
    s1_e2t = l1_out[0:4]
    s1_t2t = l1_out[4:8]

    return _layer1_tc(s1_e2t, s1_t2t, cnt_e2t, cnt_t2t,
                      W1_e2t, b1_e2t, W1_t2t, b1_t2t, Wout, bout)
